# baseline (speedup 1.0000x reference)
import numpy as np
import jax
import jax.numpy as jnp
from jax.experimental import pallas as pl
from jax.experimental.pallas import tpu as pltpu

_N = 3136
_D = 384
_M = 50000
_MB = 1000
_STEPS = _M // _MB
_NSPLIT = 2
_NB = _N // _NSPLIT
_MBC = 2048
_STEPS_C = -(-_M // _MBC)
_MPAD_C = _STEPS_C * _MBC

_PREC = jax.lax.Precision.DEFAULT


def _min_dist_kernel(e_ref, w_ref, out_ref, q_ref, en_ref):
    i = pl.program_id(1)

    @pl.when(i == 0)
    def _init():
        e = e_ref[...]
        en_ref[...] = jnp.sum(e * e, axis=1, keepdims=True)
        q_ref[...] = jnp.full((_NB, 1), jnp.inf, jnp.float32)

    w = w_ref[...]
    wn = jax.lax.dot_general(
        jnp.ones((1, _D), jnp.float32), w * w,
        (((1,), (1,)), ((), ())),
        preferred_element_type=jnp.float32,
        precision=jax.lax.Precision.HIGHEST)
    prod = jax.lax.dot_general(
        e_ref[...], w, (((1,), (1,)), ((), ())),
        preferred_element_type=jnp.float32, precision=_PREC)
    t = wn - 2.0 * prod
    q_ref[...] = jnp.minimum(q_ref[...], jnp.min(t, axis=1, keepdims=True))

    @pl.when(i == _STEPS - 1)
    def _fin():
        out_ref[...] = jnp.sqrt(
            jnp.maximum(en_ref[...] + q_ref[...], 1e-12))


def _top9_score_kernel(e_ref, w_ref, out_ref, t_ref):
    i = pl.program_id(0)
    e = e_ref[...]
    w = w_ref[...]
    ones = jnp.ones((1, _D), jnp.float32)
    wn = jax.lax.dot_general(
        ones, w * w, (((1,), (1,)), ((), ())),
        preferred_element_type=jnp.float32,
        precision=jax.lax.Precision.HIGHEST)
    prod = jax.lax.dot_general(
        e, w, (((1,), (1,)), ((), ())),
        preferred_element_type=jnp.float32, precision=_PREC)
    col = jax.lax.broadcasted_iota(jnp.int32, (1, _MBC), 1) + i * _MBC
    t = jnp.where(col < _M, wn - 2.0 * prod, jnp.float32(jnp.inf))
    t_ref[0:1, pl.ds(i * _MBC, _MBC)] = t

    @pl.when(i == _STEPS_C - 1)
    def _fin():
        en = jax.lax.dot_general(
            e, e, (((1,), (1,)), ((), ())),
            preferred_element_type=jnp.float32,
            precision=jax.lax.Precision.HIGHEST)
        row = t_ref[...]
        colv = jax.lax.broadcasted_iota(jnp.int32, (1, _MPAD_C), 1)
        cs = []
        for _ in range(9):
            m = jnp.min(row, axis=1, keepdims=True)
            pos = jnp.min(jnp.where(row == m, colv, _MPAD_C),
                          axis=1, keepdims=True)
            row = jnp.where(colv == pos, jnp.float32(jnp.inf), row)
            cs.append(jnp.sqrt(jnp.maximum(en + m, 1e-12)))
        s = cs[0] * 0.0
        for c in cs:
            s = s + jnp.exp(c)
        wgt = 1.0 - jnp.exp(cs[8]) / s
        out_ref[...] = wgt * cs[0]


def _blur_kernel(bu_ref, p_ref, out_ref):
    bu = bu_ref[...]
    p = p_ref[0]
    t1 = jax.lax.dot_general(
        bu, p, (((1,), (0,)), ((), ())),
        preferred_element_type=jnp.float32, precision=jax.lax.Precision.HIGHEST)
    t2 = jax.lax.dot_general(
        t1, bu, (((1,), (1,)), ((), ())),
        preferred_element_type=jnp.float32, precision=jax.lax.Precision.HIGHEST)
    out_ref[0, 0] = t2


def _blur_up_matrix():
    sigma = 4.0
    ksize = 33
    x = np.arange(ksize, dtype=np.float64) - (ksize - 1) / 2.0
    g = np.exp(-(x ** 2) / (2.0 * sigma * sigma))
    g /= g.sum()
    A = np.zeros((224, 224), np.float64)
    for o in range(ksize):
        for i in range(224):
            p = i + o - (ksize // 2)
            if p < 0:
                p = -p
            if p > 223:
                p = 446 - p
            A[i, p] += g[o]
    U = np.zeros((224, 28), np.float64)
    U[np.arange(224), np.arange(224) // 8] = 1.0
    return (A @ U).astype(np.float32)


_BU = _blur_up_matrix()


def kernel(embedding, memory_bank):
    row_min = pl.pallas_call(
        _min_dist_kernel,
        grid=(_NSPLIT, _STEPS),
        in_specs=[
            pl.BlockSpec((_NB, _D), lambda n, i: (n, 0)),
            pl.BlockSpec((_MB, _D), lambda n, i: (i, 0)),
        ],
        out_specs=pl.BlockSpec((_NB, 1), lambda n, i: (n, 0)),
        out_shape=jax.ShapeDtypeStruct((_N, 1), jnp.float32),
        scratch_shapes=[pltpu.VMEM((_NB, 1), jnp.float32),
                        pltpu.VMEM((_NB, 1), jnp.float32)],
        compiler_params=pltpu.CompilerParams(
            dimension_semantics=("parallel", "arbitrary")),
    )(embedding, memory_bank)

    rm = row_min[:, 0]
    idx = jnp.argmax(rm)
    erow = jax.lax.dynamic_slice(embedding, (idx, 0), (1, _D))

    score = pl.pallas_call(
        _top9_score_kernel,
        grid=(_STEPS_C,),
        in_specs=[
            pl.BlockSpec((1, _D), lambda i: (0, 0)),
            pl.BlockSpec((_MBC, _D), lambda i: (i, 0)),
        ],
        out_specs=pl.BlockSpec((1, 1), lambda i: (0, 0)),
        out_shape=jax.ShapeDtypeStruct((1, 1), jnp.float32),
        scratch_shapes=[pltpu.VMEM((1, _MPAD_C), jnp.float32)],
    )(erow, memory_bank)

    pmap = rm.reshape(4, 28, 28)
    amap = pl.pallas_call(
        _blur_kernel,
        grid=(4,),
        in_specs=[
            pl.BlockSpec((224, 28), lambda b: (0, 0)),
            pl.BlockSpec((1, 28, 28), lambda b: (b, 0, 0)),
        ],
        out_specs=pl.BlockSpec((1, 1, 224, 224), lambda b: (b, 0, 0, 0)),
        out_shape=jax.ShapeDtypeStruct((4, 1, 224, 224), jnp.float32),
        compiler_params=pltpu.CompilerParams(
            dimension_semantics=("parallel",)),
    )(jnp.asarray(_BU), pmap)

    return amap, score[0, 0]


# speedup vs baseline: 1.2157x; 1.2157x over previous
import numpy as np
import jax
import jax.numpy as jnp
from jax.experimental import pallas as pl
from jax.experimental.pallas import tpu as pltpu

_N = 3136
_D = 384
_M = 50000
_MB = 1000
_STEPS = _M // _MB
_NSPLIT = 1
_NB = _N // _NSPLIT
_MBC = 2048
_STEPS_C = -(-_M // _MBC)
_MPAD_C = _STEPS_C * _MBC

_PREC = jax.lax.Precision.DEFAULT


def _min_dist_kernel(e_ref, w_ref, out_ref, q_ref, en_ref):
    i = pl.program_id(1)

    @pl.when(i == 0)
    def _init():
        e = e_ref[...]
        en_ref[...] = jnp.sum(e * e, axis=1, keepdims=True)
        q_ref[...] = jnp.full((_NB, 1), jnp.inf, jnp.float32)

    w = w_ref[...]
    wn = jax.lax.dot_general(
        jnp.ones((1, _D), jnp.float32), w * w,
        (((1,), (1,)), ((), ())),
        preferred_element_type=jnp.float32,
        precision=jax.lax.Precision.HIGHEST)
    prod = jax.lax.dot_general(
        e_ref[...], w, (((1,), (1,)), ((), ())),
        preferred_element_type=jnp.float32, precision=_PREC)
    t = wn - 2.0 * prod
    q_ref[...] = jnp.minimum(q_ref[...], jnp.min(t, axis=1, keepdims=True))

    @pl.when(i == _STEPS - 1)
    def _fin():
        out_ref[...] = jnp.sqrt(
            jnp.maximum(en_ref[...] + q_ref[...], 1e-12))


def _top9_score_kernel(e_ref, w_ref, out_ref, t_ref):
    i = pl.program_id(0)
    e = e_ref[...]
    w = w_ref[...]
    ones = jnp.ones((1, _D), jnp.float32)
    wn = jax.lax.dot_general(
        ones, w * w, (((1,), (1,)), ((), ())),
        preferred_element_type=jnp.float32,
        precision=jax.lax.Precision.HIGHEST)
    prod = jax.lax.dot_general(
        e, w, (((1,), (1,)), ((), ())),
        preferred_element_type=jnp.float32, precision=_PREC)
    col = jax.lax.broadcasted_iota(jnp.int32, (1, _MBC), 1) + i * _MBC
    t = jnp.where(col < _M, wn - 2.0 * prod, jnp.float32(jnp.inf))
    t_ref[0:1, pl.ds(i * _MBC, _MBC)] = t

    @pl.when(i == _STEPS_C - 1)
    def _fin():
        en = jax.lax.dot_general(
            e, e, (((1,), (1,)), ((), ())),
            preferred_element_type=jnp.float32,
            precision=jax.lax.Precision.HIGHEST)
        row = t_ref[...]
        colv = jax.lax.broadcasted_iota(jnp.int32, (1, _MPAD_C), 1)
        cs = []
        for _ in range(9):
            m = jnp.min(row, axis=1, keepdims=True)
            pos = jnp.min(jnp.where(row == m, colv, _MPAD_C),
                          axis=1, keepdims=True)
            row = jnp.where(colv == pos, jnp.float32(jnp.inf), row)
            cs.append(jnp.sqrt(jnp.maximum(en + m, 1e-12)))
        s = cs[0] * 0.0
        for c in cs:
            s = s + jnp.exp(c)
        wgt = 1.0 - jnp.exp(cs[8]) / s
        out_ref[...] = wgt * cs[0]


def _blur_kernel(bu_ref, p_ref, out_ref):
    bu = bu_ref[...]
    p = p_ref[0]
    t1 = jax.lax.dot_general(
        bu, p, (((1,), (0,)), ((), ())),
        preferred_element_type=jnp.float32, precision=jax.lax.Precision.HIGHEST)
    t2 = jax.lax.dot_general(
        t1, bu, (((1,), (1,)), ((), ())),
        preferred_element_type=jnp.float32, precision=jax.lax.Precision.HIGHEST)
    out_ref[0, 0] = t2


def _blur_up_matrix():
    sigma = 4.0
    ksize = 33
    x = np.arange(ksize, dtype=np.float64) - (ksize - 1) / 2.0
    g = np.exp(-(x ** 2) / (2.0 * sigma * sigma))
    g /= g.sum()
    A = np.zeros((224, 224), np.float64)
    for o in range(ksize):
        for i in range(224):
            p = i + o - (ksize // 2)
            if p < 0:
                p = -p
            if p > 223:
                p = 446 - p
            A[i, p] += g[o]
    U = np.zeros((224, 28), np.float64)
    U[np.arange(224), np.arange(224) // 8] = 1.0
    return (A @ U).astype(np.float32)


_BU = _blur_up_matrix()


def kernel(embedding, memory_bank):
    row_min = pl.pallas_call(
        _min_dist_kernel,
        grid=(_NSPLIT, _STEPS),
        in_specs=[
            pl.BlockSpec((_NB, _D), lambda n, i: (n, 0)),
            pl.BlockSpec((_MB, _D), lambda n, i: (i, 0)),
        ],
        out_specs=pl.BlockSpec((_NB, 1), lambda n, i: (n, 0)),
        out_shape=jax.ShapeDtypeStruct((_N, 1), jnp.float32),
        scratch_shapes=[pltpu.VMEM((_NB, 1), jnp.float32),
                        pltpu.VMEM((_NB, 1), jnp.float32)],
        compiler_params=pltpu.CompilerParams(
            dimension_semantics=("parallel", "arbitrary")),
    )(embedding, memory_bank)

    rm = row_min[:, 0]
    idx = jnp.argmax(rm)
    erow = jax.lax.dynamic_slice(embedding, (idx, 0), (1, _D))

    score = pl.pallas_call(
        _top9_score_kernel,
        grid=(_STEPS_C,),
        in_specs=[
            pl.BlockSpec((1, _D), lambda i: (0, 0)),
            pl.BlockSpec((_MBC, _D), lambda i: (i, 0)),
        ],
        out_specs=pl.BlockSpec((1, 1), lambda i: (0, 0)),
        out_shape=jax.ShapeDtypeStruct((1, 1), jnp.float32),
        scratch_shapes=[pltpu.VMEM((1, _MPAD_C), jnp.float32)],
    )(erow, memory_bank)

    pmap = rm.reshape(4, 28, 28)
    amap = pl.pallas_call(
        _blur_kernel,
        grid=(4,),
        in_specs=[
            pl.BlockSpec((224, 28), lambda b: (0, 0)),
            pl.BlockSpec((1, 28, 28), lambda b: (b, 0, 0)),
        ],
        out_specs=pl.BlockSpec((1, 1, 224, 224), lambda b: (b, 0, 0, 0)),
        out_shape=jax.ShapeDtypeStruct((4, 1, 224, 224), jnp.float32),
        compiler_params=pltpu.CompilerParams(
            dimension_semantics=("parallel",)),
    )(jnp.asarray(_BU), pmap)

    return amap, score[0, 0]


# speedup vs baseline: 1.4500x; 1.1928x over previous
import numpy as np
import jax
import jax.numpy as jnp
from jax.experimental import pallas as pl
from jax.experimental.pallas import tpu as pltpu

_N = 3136
_D = 384
_M = 50000
_MB = 1000
_STEPS = _M // _MB
_MBC = 2048
_STEPS_C = -(-_M // _MBC)
_MPAD_C = _STEPS_C * _MBC

_PREC = jax.lax.Precision.DEFAULT


def _min_dist_kernel(e32_ref, ebf_ref, w_ref, out_ref, q_ref, en_ref):
    i = pl.program_id(0)

    @pl.when(i == 0)
    def _init():
        e = e32_ref[...]
        en_ref[...] = jax.lax.dot_general(
            jnp.ones((1, _D), jnp.float32), e * e,
            (((1,), (1,)), ((), ())),
            preferred_element_type=jnp.float32,
            precision=jax.lax.Precision.HIGHEST)
        q_ref[...] = jnp.full((1, _N), jnp.inf, jnp.float32)

    w = w_ref[...]
    wn = jnp.sum(w * w, axis=1, keepdims=True)
    prod = jax.lax.dot_general(
        w.astype(jnp.bfloat16), ebf_ref[...], (((1,), (1,)), ((), ())),
        preferred_element_type=jnp.float32, precision=_PREC)
    t = wn - 2.0 * prod
    q_ref[...] = jnp.minimum(q_ref[...], jnp.min(t, axis=0, keepdims=True))

    @pl.when(i == _STEPS - 1)
    def _fin():
        out_ref[...] = jnp.sqrt(
            jnp.maximum(en_ref[...] + q_ref[...], 1e-12))


def _top9_score_kernel(e_ref, w_ref, out_ref, t_ref):
    i = pl.program_id(0)
    e = e_ref[...]
    w = w_ref[...]
    ones = jnp.ones((1, _D), jnp.float32)
    wn = jax.lax.dot_general(
        ones, w * w, (((1,), (1,)), ((), ())),
        preferred_element_type=jnp.float32,
        precision=jax.lax.Precision.HIGHEST)
    prod = jax.lax.dot_general(
        e, w, (((1,), (1,)), ((), ())),
        preferred_element_type=jnp.float32, precision=_PREC)
    col = jax.lax.broadcasted_iota(jnp.int32, (1, _MBC), 1) + i * _MBC
    t = jnp.where(col < _M, wn - 2.0 * prod, jnp.float32(jnp.inf))
    t_ref[0:1, pl.ds(i * _MBC, _MBC)] = t

    @pl.when(i == _STEPS_C - 1)
    def _fin():
        en = jax.lax.dot_general(
            e, e, (((1,), (1,)), ((), ())),
            preferred_element_type=jnp.float32,
            precision=jax.lax.Precision.HIGHEST)
        row = t_ref[...]
        colv = jax.lax.broadcasted_iota(jnp.int32, (1, _MPAD_C), 1)
        cs = []
        for _ in range(9):
            m = jnp.min(row, axis=1, keepdims=True)
            pos = jnp.min(jnp.where(row == m, colv, _MPAD_C),
                          axis=1, keepdims=True)
            row = jnp.where(colv == pos, jnp.float32(jnp.inf), row)
            cs.append(jnp.sqrt(jnp.maximum(en + m, 1e-12)))
        s = cs[0] * 0.0
        for c in cs:
            s = s + jnp.exp(c)
        wgt = 1.0 - jnp.exp(cs[8]) / s
        out_ref[...] = wgt * cs[0]


def _blur_kernel(bu_ref, p_ref, out_ref):
    bu = bu_ref[...]
    p = p_ref[0]
    t1 = jax.lax.dot_general(
        bu, p, (((1,), (0,)), ((), ())),
        preferred_element_type=jnp.float32, precision=jax.lax.Precision.HIGHEST)
    t2 = jax.lax.dot_general(
        t1, bu, (((1,), (1,)), ((), ())),
        preferred_element_type=jnp.float32, precision=jax.lax.Precision.HIGHEST)
    out_ref[0, 0] = t2


def _blur_up_matrix():
    sigma = 4.0
    ksize = 33
    x = np.arange(ksize, dtype=np.float64) - (ksize - 1) / 2.0
    g = np.exp(-(x ** 2) / (2.0 * sigma * sigma))
    g /= g.sum()
    A = np.zeros((224, 224), np.float64)
    for o in range(ksize):
        for i in range(224):
            p = i + o - (ksize // 2)
            if p < 0:
                p = -p
            if p > 223:
                p = 446 - p
            A[i, p] += g[o]
    U = np.zeros((224, 28), np.float64)
    U[np.arange(224), np.arange(224) // 8] = 1.0
    return (A @ U).astype(np.float32)


_BU = _blur_up_matrix()


def kernel(embedding, memory_bank):
    row_min = pl.pallas_call(
        _min_dist_kernel,
        grid=(_STEPS,),
        in_specs=[
            pl.BlockSpec((_N, _D), lambda i: (0, 0)),
            pl.BlockSpec((_N, _D), lambda i: (0, 0)),
            pl.BlockSpec((_MB, _D), lambda i: (i, 0)),
        ],
        out_specs=pl.BlockSpec((1, _N), lambda i: (0, 0)),
        out_shape=jax.ShapeDtypeStruct((1, _N), jnp.float32),
        scratch_shapes=[pltpu.VMEM((1, _N), jnp.float32),
                        pltpu.VMEM((1, _N), jnp.float32)],
    )(embedding, embedding.astype(jnp.bfloat16), memory_bank)

    rm = row_min[0]
    idx = jnp.argmax(rm)
    erow = jax.lax.dynamic_slice(embedding, (idx, 0), (1, _D))

    score = pl.pallas_call(
        _top9_score_kernel,
        grid=(_STEPS_C,),
        in_specs=[
            pl.BlockSpec((1, _D), lambda i: (0, 0)),
            pl.BlockSpec((_MBC, _D), lambda i: (i, 0)),
        ],
        out_specs=pl.BlockSpec((1, 1), lambda i: (0, 0)),
        out_shape=jax.ShapeDtypeStruct((1, 1), jnp.float32),
        scratch_shapes=[pltpu.VMEM((1, _MPAD_C), jnp.float32)],
    )(erow, memory_bank)

    pmap = rm.reshape(4, 28, 28)
    amap = pl.pallas_call(
        _blur_kernel,
        grid=(4,),
        in_specs=[
            pl.BlockSpec((224, 28), lambda b: (0, 0)),
            pl.BlockSpec((1, 28, 28), lambda b: (b, 0, 0)),
        ],
        out_specs=pl.BlockSpec((1, 1, 224, 224), lambda b: (b, 0, 0, 0)),
        out_shape=jax.ShapeDtypeStruct((4, 1, 224, 224), jnp.float32),
        compiler_params=pltpu.CompilerParams(
            dimension_semantics=("parallel",)),
    )(jnp.asarray(_BU), pmap)

    return amap, score[0, 0]


# speedup vs baseline: 1.4906x; 1.0280x over previous
import numpy as np
import jax
import jax.numpy as jnp
from jax.experimental import pallas as pl
from jax.experimental.pallas import tpu as pltpu

_N = 3136
_D = 384
_M = 50000
_MB = 1000
_STEPS = _M // _MB
_MBC = 2048
_STEPS_C = -(-_M // _MBC)
_MPAD_C = _STEPS_C * _MBC

_PREC = jax.lax.Precision.DEFAULT


def _min_dist_kernel(e_ref, w_ref, out_ref, q_ref, en_ref):
    i = pl.program_id(0)

    @pl.when(i == 0)
    def _init():
        e = e_ref[...]
        en_ref[...] = jax.lax.dot_general(
            jnp.ones((1, _D), jnp.float32), e * e,
            (((1,), (1,)), ((), ())),
            preferred_element_type=jnp.float32,
            precision=jax.lax.Precision.HIGHEST)
        q_ref[...] = jnp.full((1, _N), jnp.inf, jnp.float32)

    w = w_ref[...]
    wn_half = 0.5 * jnp.sum(w * w, axis=1, keepdims=True)
    prod = jax.lax.dot_general(
        w, e_ref[...], (((1,), (1,)), ((), ())),
        preferred_element_type=jnp.float32, precision=_PREC)
    t = wn_half - prod
    q_ref[...] = jnp.minimum(q_ref[...], jnp.min(t, axis=0, keepdims=True))

    @pl.when(i == _STEPS - 1)
    def _fin():
        out_ref[...] = jnp.sqrt(
            jnp.maximum(en_ref[...] + 2.0 * q_ref[...], 1e-12))


def _top9_score_kernel(e_ref, w_ref, out_ref, t_ref):
    i = pl.program_id(0)
    e = e_ref[...]
    w = w_ref[...]
    ones = jnp.ones((1, _D), jnp.float32)
    wn = jax.lax.dot_general(
        ones, w * w, (((1,), (1,)), ((), ())),
        preferred_element_type=jnp.float32,
        precision=jax.lax.Precision.HIGHEST)
    prod = jax.lax.dot_general(
        e, w, (((1,), (1,)), ((), ())),
        preferred_element_type=jnp.float32, precision=_PREC)
    col = jax.lax.broadcasted_iota(jnp.int32, (1, _MBC), 1) + i * _MBC
    t = jnp.where(col < _M, wn - 2.0 * prod, jnp.float32(jnp.inf))
    t_ref[0:1, pl.ds(i * _MBC, _MBC)] = t

    @pl.when(i == _STEPS_C - 1)
    def _fin():
        en = jax.lax.dot_general(
            e, e, (((1,), (1,)), ((), ())),
            preferred_element_type=jnp.float32,
            precision=jax.lax.Precision.HIGHEST)
        row = t_ref[...]
        colv = jax.lax.broadcasted_iota(jnp.int32, (1, _MPAD_C), 1)
        cs = []
        for _ in range(9):
            m = jnp.min(row, axis=1, keepdims=True)
            pos = jnp.min(jnp.where(row == m, colv, _MPAD_C),
                          axis=1, keepdims=True)
            row = jnp.where(colv == pos, jnp.float32(jnp.inf), row)
            cs.append(jnp.sqrt(jnp.maximum(en + m, 1e-12)))
        s = cs[0] * 0.0
        for c in cs:
            s = s + jnp.exp(c)
        wgt = 1.0 - jnp.exp(cs[8]) / s
        out_ref[...] = wgt * cs[0]


def _blur_kernel(bu_ref, p_ref, out_ref):
    bu = bu_ref[...]
    p = p_ref[0]
    t1 = jax.lax.dot_general(
        bu, p, (((1,), (0,)), ((), ())),
        preferred_element_type=jnp.float32, precision=jax.lax.Precision.HIGHEST)
    t2 = jax.lax.dot_general(
        t1, bu, (((1,), (1,)), ((), ())),
        preferred_element_type=jnp.float32, precision=jax.lax.Precision.HIGHEST)
    out_ref[0, 0] = t2


def _blur_up_matrix():
    sigma = 4.0
    ksize = 33
    x = np.arange(ksize, dtype=np.float64) - (ksize - 1) / 2.0
    g = np.exp(-(x ** 2) / (2.0 * sigma * sigma))
    g /= g.sum()
    A = np.zeros((224, 224), np.float64)
    for o in range(ksize):
        for i in range(224):
            p = i + o - (ksize // 2)
            if p < 0:
                p = -p
            if p > 223:
                p = 446 - p
            A[i, p] += g[o]
    U = np.zeros((224, 28), np.float64)
    U[np.arange(224), np.arange(224) // 8] = 1.0
    return (A @ U).astype(np.float32)


_BU = _blur_up_matrix()


def kernel(embedding, memory_bank):
    row_min = pl.pallas_call(
        _min_dist_kernel,
        grid=(_STEPS,),
        in_specs=[
            pl.BlockSpec((_N, _D), lambda i: (0, 0)),
            pl.BlockSpec((_MB, _D), lambda i: (i, 0)),
        ],
        out_specs=pl.BlockSpec((1, _N), lambda i: (0, 0)),
        out_shape=jax.ShapeDtypeStruct((1, _N), jnp.float32),
        scratch_shapes=[pltpu.VMEM((1, _N), jnp.float32),
                        pltpu.VMEM((1, _N), jnp.float32)],
    )(embedding, memory_bank)

    rm = row_min[0]
    idx = jnp.argmax(rm)
    erow = jax.lax.dynamic_slice(embedding, (idx, 0), (1, _D))

    score = pl.pallas_call(
        _top9_score_kernel,
        grid=(_STEPS_C,),
        in_specs=[
            pl.BlockSpec((1, _D), lambda i: (0, 0)),
            pl.BlockSpec((_MBC, _D), lambda i: (i, 0)),
        ],
        out_specs=pl.BlockSpec((1, 1), lambda i: (0, 0)),
        out_shape=jax.ShapeDtypeStruct((1, 1), jnp.float32),
        scratch_shapes=[pltpu.VMEM((1, _MPAD_C), jnp.float32)],
    )(erow, memory_bank)

    pmap = rm.reshape(4, 28, 28)
    amap = pl.pallas_call(
        _blur_kernel,
        grid=(4,),
        in_specs=[
            pl.BlockSpec((224, 28), lambda b: (0, 0)),
            pl.BlockSpec((1, 28, 28), lambda b: (b, 0, 0)),
        ],
        out_specs=pl.BlockSpec((1, 1, 224, 224), lambda b: (b, 0, 0, 0)),
        out_shape=jax.ShapeDtypeStruct((4, 1, 224, 224), jnp.float32),
        compiler_params=pltpu.CompilerParams(
            dimension_semantics=("parallel",)),
    )(jnp.asarray(_BU), pmap)

    return amap, score[0, 0]


# speedup vs baseline: 1.7890x; 1.2002x over previous
import numpy as np
import jax
import jax.numpy as jnp
from jax.experimental import pallas as pl
from jax.experimental.pallas import tpu as pltpu

_N = 3136
_D = 384
_M = 50000
_MB = 1000
_STEPS = _M // _MB
_MBC = 8192
_STEPS_C = -(-_M // _MBC)
_MPAD_C = _STEPS_C * _MBC

_PREC = jax.lax.Precision.DEFAULT


def _min_dist_kernel(e_ref, w_ref, out_ref, q_ref, en_ref):
    i = pl.program_id(0)

    @pl.when(i == 0)
    def _init():
        e = e_ref[...]
        en_ref[...] = jax.lax.dot_general(
            jnp.ones((1, _D), jnp.float32), e * e,
            (((1,), (1,)), ((), ())),
            preferred_element_type=jnp.float32,
            precision=jax.lax.Precision.HIGHEST)
        q_ref[...] = jnp.full((1, _N), jnp.inf, jnp.float32)

    w = w_ref[...]
    wn_half = 0.5 * jnp.sum(w * w, axis=1, keepdims=True)
    prod = jax.lax.dot_general(
        w, e_ref[...], (((1,), (1,)), ((), ())),
        preferred_element_type=jnp.float32, precision=_PREC)
    t = wn_half - prod
    q_ref[...] = jnp.minimum(q_ref[...], jnp.min(t, axis=0, keepdims=True))

    @pl.when(i == _STEPS - 1)
    def _fin():
        out_ref[...] = jnp.sqrt(
            jnp.maximum(en_ref[...] + 2.0 * q_ref[...], 1e-12))


def _top9_score_kernel(e_ref, w_ref, out_ref, t_ref):
    i = pl.program_id(0)
    ea = e_ref[...]
    w = w_ref[...]
    wn_half = 0.5 * jnp.sum(w * w, axis=1, keepdims=True)
    wn_hi = wn_half.astype(jnp.bfloat16).astype(jnp.float32)
    wn_lo = wn_half - wn_hi
    waug = jnp.concatenate([w, wn_hi, wn_lo], axis=1)
    prod = jax.lax.dot_general(
        ea, waug, (((1,), (1,)), ((), ())),
        preferred_element_type=jnp.float32, precision=_PREC)
    col = jax.lax.broadcasted_iota(jnp.int32, (1, _MBC), 1) + i * _MBC
    t = jnp.where(col < _M, -2.0 * prod, jnp.float32(jnp.inf))
    t_ref[0:1, pl.ds(i * _MBC, _MBC)] = t

    @pl.when(i == _STEPS_C - 1)
    def _fin():
        en = jax.lax.dot_general(
            ea, ea, (((1,), (1,)), ((), ())),
            preferred_element_type=jnp.float32,
            precision=jax.lax.Precision.HIGHEST) - 2.0
        row = t_ref[...]
        colv = jax.lax.broadcasted_iota(jnp.int32, (1, _MPAD_C), 1)
        cs = []
        for _ in range(9):
            m = jnp.min(row, axis=1, keepdims=True)
            pos = jnp.min(jnp.where(row == m, colv, _MPAD_C),
                          axis=1, keepdims=True)
            row = jnp.where(colv == pos, jnp.float32(jnp.inf), row)
            cs.append(jnp.sqrt(jnp.maximum(en + m, 1e-12)))
        s = cs[0] * 0.0
        for c in cs:
            s = s + jnp.exp(c)
        wgt = 1.0 - jnp.exp(cs[8]) / s
        out_ref[...] = wgt * cs[0]


def _blur_kernel(bu_ref, p_ref, out_ref):
    bu = bu_ref[...]
    p = p_ref[0]
    t1 = jax.lax.dot_general(
        bu, p, (((1,), (0,)), ((), ())),
        preferred_element_type=jnp.float32, precision=jax.lax.Precision.HIGHEST)
    t2 = jax.lax.dot_general(
        t1, bu, (((1,), (1,)), ((), ())),
        preferred_element_type=jnp.float32, precision=jax.lax.Precision.HIGHEST)
    out_ref[0, 0] = t2


def _blur_up_matrix():
    sigma = 4.0
    ksize = 33
    x = np.arange(ksize, dtype=np.float64) - (ksize - 1) / 2.0
    g = np.exp(-(x ** 2) / (2.0 * sigma * sigma))
    g /= g.sum()
    A = np.zeros((224, 224), np.float64)
    for o in range(ksize):
        for i in range(224):
            p = i + o - (ksize // 2)
            if p < 0:
                p = -p
            if p > 223:
                p = 446 - p
            A[i, p] += g[o]
    U = np.zeros((224, 28), np.float64)
    U[np.arange(224), np.arange(224) // 8] = 1.0
    return (A @ U).astype(np.float32)


_BU = _blur_up_matrix()


def kernel(embedding, memory_bank):
    row_min = pl.pallas_call(
        _min_dist_kernel,
        grid=(_STEPS,),
        in_specs=[
            pl.BlockSpec((_N, _D), lambda i: (0, 0)),
            pl.BlockSpec((_MB, _D), lambda i: (i, 0)),
        ],
        out_specs=pl.BlockSpec((1, _N), lambda i: (0, 0)),
        out_shape=jax.ShapeDtypeStruct((1, _N), jnp.float32),
        scratch_shapes=[pltpu.VMEM((1, _N), jnp.float32),
                        pltpu.VMEM((1, _N), jnp.float32)],
    )(embedding, memory_bank)

    rm = row_min[0]
    idx = jnp.argmax(rm)
    erow = jax.lax.dynamic_slice(embedding, (idx, 0), (1, _D))

    eaug = jnp.concatenate(
        [erow, jnp.full((1, 2), -1.0, jnp.float32)], axis=1)
    score = pl.pallas_call(
        _top9_score_kernel,
        grid=(_STEPS_C,),
        in_specs=[
            pl.BlockSpec((1, _D + 2), lambda i: (0, 0)),
            pl.BlockSpec((_MBC, _D), lambda i: (i, 0)),
        ],
        out_specs=pl.BlockSpec((1, 1), lambda i: (0, 0)),
        out_shape=jax.ShapeDtypeStruct((1, 1), jnp.float32),
        scratch_shapes=[pltpu.VMEM((1, _MPAD_C), jnp.float32)],
    )(eaug, memory_bank)

    pmap = rm.reshape(4, 28, 28)
    amap = pl.pallas_call(
        _blur_kernel,
        grid=(4,),
        in_specs=[
            pl.BlockSpec((224, 28), lambda b: (0, 0)),
            pl.BlockSpec((1, 28, 28), lambda b: (b, 0, 0)),
        ],
        out_specs=pl.BlockSpec((1, 1, 224, 224), lambda b: (b, 0, 0, 0)),
        out_shape=jax.ShapeDtypeStruct((4, 1, 224, 224), jnp.float32),
        compiler_params=pltpu.CompilerParams(
            dimension_semantics=("parallel",)),
    )(jnp.asarray(_BU), pmap)

    return amap, score[0, 0]


# speedup vs baseline: 1.8979x; 1.0609x over previous
import numpy as np
import jax
import jax.numpy as jnp
from jax.experimental import pallas as pl
from jax.experimental.pallas import tpu as pltpu

_N = 3136
_D = 384
_M = 50000
_MB = 2000
_STEPS = _M // _MB
_MBC = 8192
_STEPS_C = -(-_M // _MBC)
_MPAD_C = _STEPS_C * _MBC

_PREC = jax.lax.Precision.DEFAULT


def _min_dist_kernel(e_ref, w_ref, out_ref, q_ref, en_ref):
    i = pl.program_id(0)

    @pl.when(i == 0)
    def _init():
        e = e_ref[...]
        en_ref[...] = jax.lax.dot_general(
            jnp.ones((1, _D), jnp.float32), e * e,
            (((1,), (1,)), ((), ())),
            preferred_element_type=jnp.float32,
            precision=jax.lax.Precision.HIGHEST)
        q_ref[...] = jnp.full((1, _N), jnp.inf, jnp.float32)

    w = w_ref[...]
    wn_half = 0.5 * jnp.sum(w * w, axis=1, keepdims=True)
    prod = jax.lax.dot_general(
        w, e_ref[...], (((1,), (1,)), ((), ())),
        preferred_element_type=jnp.float32, precision=_PREC)
    t = wn_half - prod
    q_ref[...] = jnp.minimum(q_ref[...], jnp.min(t, axis=0, keepdims=True))

    @pl.when(i == _STEPS - 1)
    def _fin():
        out_ref[...] = jnp.sqrt(
            jnp.maximum(en_ref[...] + 2.0 * q_ref[...], 1e-12))


def _top9_score_kernel(e_ref, w_ref, out_ref, t_ref):
    i = pl.program_id(0)
    ea = e_ref[...]
    w = w_ref[...]
    wn_half = 0.5 * jnp.sum(w * w, axis=1, keepdims=True)
    wn_hi = wn_half.astype(jnp.bfloat16).astype(jnp.float32)
    wn_lo = wn_half - wn_hi
    waug = jnp.concatenate([w, wn_hi, wn_lo], axis=1)
    prod = jax.lax.dot_general(
        ea, waug, (((1,), (1,)), ((), ())),
        preferred_element_type=jnp.float32, precision=_PREC)
    col = jax.lax.broadcasted_iota(jnp.int32, (1, _MBC), 1) + i * _MBC
    t = jnp.where(col < _M, -2.0 * prod, jnp.float32(jnp.inf))
    t_ref[0:1, pl.ds(i * _MBC, _MBC)] = t

    @pl.when(i == _STEPS_C - 1)
    def _fin():
        en = jax.lax.dot_general(
            ea, ea, (((1,), (1,)), ((), ())),
            preferred_element_type=jnp.float32,
            precision=jax.lax.Precision.HIGHEST) - 2.0
        row = t_ref[...]
        colv = jax.lax.broadcasted_iota(jnp.int32, (1, _MPAD_C), 1)
        cs = []
        for _ in range(9):
            m = jnp.min(row, axis=1, keepdims=True)
            pos = jnp.min(jnp.where(row == m, colv, _MPAD_C),
                          axis=1, keepdims=True)
            row = jnp.where(colv == pos, jnp.float32(jnp.inf), row)
            cs.append(jnp.sqrt(jnp.maximum(en + m, 1e-12)))
        s = cs[0] * 0.0
        for c in cs:
            s = s + jnp.exp(c)
        wgt = 1.0 - jnp.exp(cs[8]) / s
        out_ref[...] = wgt * cs[0]


def _blur_kernel(bu_ref, p_ref, out_ref):
    bu = bu_ref[...]
    p = p_ref[0]
    t1 = jax.lax.dot_general(
        bu, p, (((1,), (0,)), ((), ())),
        preferred_element_type=jnp.float32, precision=jax.lax.Precision.HIGHEST)
    t2 = jax.lax.dot_general(
        t1, bu, (((1,), (1,)), ((), ())),
        preferred_element_type=jnp.float32, precision=jax.lax.Precision.HIGHEST)
    out_ref[0, 0] = t2


def _blur_up_matrix():
    sigma = 4.0
    ksize = 33
    x = np.arange(ksize, dtype=np.float64) - (ksize - 1) / 2.0
    g = np.exp(-(x ** 2) / (2.0 * sigma * sigma))
    g /= g.sum()
    A = np.zeros((224, 224), np.float64)
    for o in range(ksize):
        for i in range(224):
            p = i + o - (ksize // 2)
            if p < 0:
                p = -p
            if p > 223:
                p = 446 - p
            A[i, p] += g[o]
    U = np.zeros((224, 28), np.float64)
    U[np.arange(224), np.arange(224) // 8] = 1.0
    return (A @ U).astype(np.float32)


_BU = _blur_up_matrix()


def kernel(embedding, memory_bank):
    row_min = pl.pallas_call(
        _min_dist_kernel,
        grid=(_STEPS,),
        in_specs=[
            pl.BlockSpec((_N, _D), lambda i: (0, 0)),
            pl.BlockSpec((_MB, _D), lambda i: (i, 0)),
        ],
        out_specs=pl.BlockSpec((1, _N), lambda i: (0, 0)),
        out_shape=jax.ShapeDtypeStruct((1, _N), jnp.float32),
        scratch_shapes=[pltpu.VMEM((1, _N), jnp.float32),
                        pltpu.VMEM((1, _N), jnp.float32)],
    )(embedding, memory_bank)

    rm = row_min[0]
    idx = jnp.argmax(rm)
    erow = jax.lax.dynamic_slice(embedding, (idx, 0), (1, _D))

    eaug = jnp.concatenate(
        [erow, jnp.full((1, 2), -1.0, jnp.float32)], axis=1)
    score = pl.pallas_call(
        _top9_score_kernel,
        grid=(_STEPS_C,),
        in_specs=[
            pl.BlockSpec((1, _D + 2), lambda i: (0, 0)),
            pl.BlockSpec((_MBC, _D), lambda i: (i, 0)),
        ],
        out_specs=pl.BlockSpec((1, 1), lambda i: (0, 0)),
        out_shape=jax.ShapeDtypeStruct((1, 1), jnp.float32),
        scratch_shapes=[pltpu.VMEM((1, _MPAD_C), jnp.float32)],
    )(eaug, memory_bank)

    pmap = rm.reshape(4, 28, 28)
    amap = pl.pallas_call(
        _blur_kernel,
        grid=(4,),
        in_specs=[
            pl.BlockSpec((224, 28), lambda b: (0, 0)),
            pl.BlockSpec((1, 28, 28), lambda b: (b, 0, 0)),
        ],
        out_specs=pl.BlockSpec((1, 1, 224, 224), lambda b: (b, 0, 0, 0)),
        out_shape=jax.ShapeDtypeStruct((4, 1, 224, 224), jnp.float32),
        compiler_params=pltpu.CompilerParams(
            dimension_semantics=("parallel",)),
    )(jnp.asarray(_BU), pmap)

    return amap, score[0, 0]
